# baseline (speedup 1.0000x reference)
import functools

import jax
import jax.numpy as jnp
from jax import lax
from jax.experimental import pallas as pl
from jax.experimental.pallas import tpu as pltpu
from jax.experimental.pallas import tpu_sc as plsc

_N_BINS = 15
_N_COLS = 1000
_LANES = 16
_NW = 32
_TC_BLOCK_R = 2048
_SC_ROWS = 4096
_SC_BATCH = 16
_STEP = 1.0 / 15.0


def _bin_bounds_2d(n_lanes):
    idx = lax.broadcasted_iota(jnp.int32, (1, n_lanes), 1)
    idx_f = idx.astype(jnp.float32)
    step = jnp.float32(_STEP)
    lowers = jnp.where(idx < _N_BINS, idx_f * step, jnp.float32(2.0))
    uppers = jnp.where(idx == _N_BINS - 1, jnp.float32(1.0),
                       (idx_f + 1.0) * step)
    return lowers, uppers


def _histogram(conf, acc):
    lowers, uppers = _bin_bounds_2d(_LANES)
    in_bin = ((conf > lowers) & (conf <= uppers)).astype(jnp.float32)
    counts = jnp.sum(in_bin, axis=0, keepdims=True)
    acc_sums = jnp.sum(acc * in_bin, axis=0, keepdims=True)
    conf_sums = jnp.sum(conf * in_bin, axis=0, keepdims=True)
    return jnp.concatenate([counts, acc_sums, conf_sums], axis=0)


def _tc_kernel(logits_ref, labels_ref, part_ref, acc_scratch, *, block_r):
    i = pl.program_id(0)
    g = pl.num_programs(0)

    x = logits_ref[...]
    m = jnp.max(x, axis=1, keepdims=True)
    s = jnp.sum(jnp.exp(x - m), axis=1, keepdims=True)
    conf = 1.0 / s
    lab = labels_ref[0, 0, :].reshape(block_r, 1)
    cols = lax.broadcasted_iota(jnp.int32, x.shape, 1)
    xlab = jnp.max(jnp.where(cols == lab, x, -jnp.inf), axis=1, keepdims=True)
    acc = (xlab == m).astype(jnp.float32)

    partial = _histogram(conf, acc)

    @pl.when(i == 0)
    def _init():
        acc_scratch[...] = partial

    @pl.when(i != 0)
    def _accum():
        acc_scratch[...] = acc_scratch[...] + partial

    @pl.when(i == g - 1)
    def _emit():
        part_ref[...] = acc_scratch[...]


def _tc_partials(logits, labels, tc_rows):
    grid = tc_rows // _TC_BLOCK_R
    labels3 = labels.reshape(labels.shape[0] // _TC_BLOCK_R, 1, _TC_BLOCK_R)
    return pl.pallas_call(
        functools.partial(_tc_kernel, block_r=_TC_BLOCK_R),
        grid=(grid,),
        in_specs=[
            pl.BlockSpec((_TC_BLOCK_R, _N_COLS), lambda i: (i, 0)),
            pl.BlockSpec((1, 1, _TC_BLOCK_R), lambda i: (i, 0, 0)),
        ],
        out_specs=pl.BlockSpec((3, _LANES), lambda i: (0, 0)),
        out_shape=jax.ShapeDtypeStruct((3, _LANES), jnp.float32),
        scratch_shapes=[pltpu.VMEM((3, _LANES), jnp.float32)],
        compiler_params=pltpu.CompilerParams(
            allow_input_fusion=[True, True]),
    )(logits, labels3)


def _sc_body(logits_hbm, parts_hbm, buf_v, stage_m, *, sc_base, rpw):
    nc = 2
    wid = lax.axis_index("s") * nc + lax.axis_index("c")
    row0 = sc_base + wid * rpw
    out0 = wid * rpw

    lane = lax.iota(jnp.int32, _LANES)
    lane_f = lane.astype(jnp.float32)
    n_full = _N_COLS // _LANES
    tail0 = _N_COLS - _LANES
    neg_inf = jnp.full((_LANES,), -jnp.inf, jnp.float32)
    zero = jnp.zeros((_LANES,), jnp.float32)

    def batch_fn(b, _):
        pltpu.sync_copy(
            logits_hbm.at[pl.ds(row0 + b * _SC_BATCH, _SC_BATCH)], buf_v)

        def row_fn(j, __):
            m0 = neg_inf
            m1 = neg_inf
            s0 = zero
            s1 = zero
            a0 = zero
            a1 = zero
            for c in range(0, n_full, 2):
                x0 = buf_v[j, pl.ds(c * _LANES, _LANES)]
                x1 = buf_v[j, pl.ds((c + 1) * _LANES, _LANES)]
                a0 = jnp.where(x0 > m0, lane_f + float(c * _LANES), a0)
                a1 = jnp.where(x1 > m1, lane_f + float((c + 1) * _LANES), a1)
                m0 = jnp.maximum(m0, x0)
                m1 = jnp.maximum(m1, x1)
                s0 = s0 + jnp.exp(x0)
                s1 = s1 + jnp.exp(x1)
            xt = buf_v[j, pl.ds(tail0, _LANES)]
            a0 = jnp.where(xt > m0, lane_f + float(tail0), a0)
            m0 = jnp.maximum(m0, xt)
            s0 = s0 + jnp.where(lane >= n_full * _LANES - tail0,
                                jnp.exp(xt), jnp.float32(0.0))
            mm = jnp.maximum(m0, m1)
            cand0 = jnp.where(m0 == mm, a0, jnp.float32(1e9))
            cand1 = jnp.where(m1 == mm, a1, jnp.float32(1e9))
            stage_m[j, pl.ds(0, _LANES)] = mm
            stage_m[j, pl.ds(_LANES, _LANES)] = s0 + s1
            stage_m[j, pl.ds(2 * _LANES, _LANES)] = jnp.minimum(cand0, cand1)
            return 0

        lax.fori_loop(0, _SC_BATCH, row_fn, 0)
        pltpu.sync_copy(
            stage_m, parts_hbm.at[pl.ds(out0 + b * _SC_BATCH, _SC_BATCH)])
        return 0

    lax.fori_loop(0, rpw // _SC_BATCH, batch_fn, 0)


def _sc_partials(logits):
    sc_base = 0
    rpw = _SC_ROWS // _NW
    mesh = plsc.VectorSubcoreMesh(core_axis_name="c", subcore_axis_name="s")
    run = pl.kernel(
        functools.partial(_sc_body, sc_base=sc_base, rpw=rpw),
        mesh=mesh,
        out_type=jax.ShapeDtypeStruct((_SC_ROWS, 3 * _LANES), jnp.float32),
        scratch_types=[
            pltpu.VMEM((_SC_BATCH, _N_COLS), jnp.float32),
            pltpu.VMEM((_SC_BATCH, 3 * _LANES), jnp.float32),
        ],
        compiler_params=pltpu.CompilerParams(use_tc_tiling_on_sc=True),
    )
    return run(logits)


def _combine_kernel(tc_ref, parts_ref, labs_ref, ece_ref, accs_ref,
                    confs_ref, *, n_rows):
    p = parts_ref[...]
    maxes = p[:, 0:_LANES]
    mx = jnp.max(maxes, axis=1, keepdims=True)
    s = jnp.sum(p[:, _LANES:2 * _LANES], axis=1, keepdims=True)
    conf = jnp.exp(mx) / s
    args = p[:, 2 * _LANES:3 * _LANES]
    pred = jnp.min(jnp.where(maxes == mx, args, jnp.float32(1e9)),
                   axis=1, keepdims=True)
    acc = (pred == labs_ref[...].astype(jnp.float32)).astype(jnp.float32)

    tot = tc_ref[...] + _histogram(conf, acc)
    count = tot[0:1, :_N_BINS]
    acc_sum = tot[1:2, :_N_BINS]
    conf_sum = tot[2:3, :_N_BINS]
    prop = count / float(n_rows)
    safe = jnp.maximum(count, 1.0)
    acc_bin = acc_sum / safe
    conf_bin = conf_sum / safe
    nonempty = count > 0.0
    gaps = jnp.where(nonempty, jnp.abs(conf_bin - acc_bin) * prop, 0.0)
    ece_ref[...] = jnp.sum(gaps, keepdims=True)
    accs_ref[...] = jnp.where(nonempty, acc_bin, jnp.nan)
    confs_ref[...] = jnp.where(nonempty, conf_bin, jnp.nan)


@jax.jit
def kernel(logits, labels):
    n_rows, _ = logits.shape
    tc_rows = n_rows - _SC_ROWS
    sc_parts = _sc_partials(logits[tc_rows:])
    tc_part = _tc_partials(logits, labels, tc_rows)
    labs_sc = labels[tc_rows:].reshape(_SC_ROWS, 1)
    ece, accs, confs = pl.pallas_call(
        functools.partial(_combine_kernel, n_rows=n_rows),
        out_shape=[
            jax.ShapeDtypeStruct((1, 1), jnp.float32),
            jax.ShapeDtypeStruct((1, _N_BINS), jnp.float32),
            jax.ShapeDtypeStruct((1, _N_BINS), jnp.float32),
        ],
    )(tc_part, sc_parts, labs_sc)
    return ece.reshape(1), accs.reshape(_N_BINS), confs.reshape(_N_BINS)


# speedup vs baseline: 3.8256x; 3.8256x over previous
import functools

import jax
import jax.numpy as jnp
from jax import lax
from jax.experimental import pallas as pl
from jax.experimental.pallas import tpu as pltpu

_N_BINS = 15
_N_COLS = 1000
_BLOCK_C = 2048


def _tc_kernel(lt_ref, labels_ref, ece_ref, accs_ref, confs_ref,
               acc_scratch, *, n_rows, block_c):
    i = pl.program_id(0)
    g = pl.num_programs(0)

    x = lt_ref[...]
    m = jnp.max(x, axis=0, keepdims=True)
    s = jnp.sum(jnp.exp(x - m), axis=0, keepdims=True)
    conf = 1.0 / s
    lab = labels_ref[0, 0, :].reshape(1, block_c)
    rows = lax.broadcasted_iota(jnp.int32, x.shape, 0)
    xlab = jnp.max(jnp.where(rows == lab, x, -jnp.inf), axis=0, keepdims=True)
    acc = (xlab == m).astype(jnp.float32)

    idx = lax.broadcasted_iota(jnp.int32, (_N_BINS, 1), 0)
    idx_f = idx.astype(jnp.float32)
    step = jnp.float32(1.0) / jnp.float32(_N_BINS)
    lowers = idx_f * step
    uppers = jnp.where(idx == _N_BINS - 1, jnp.float32(1.0),
                       (idx_f + 1.0) * step)
    in_bin = ((conf > lowers) & (conf <= uppers)).astype(jnp.float32)
    counts = jnp.sum(in_bin, axis=1, keepdims=True)
    acc_sums = jnp.sum(acc * in_bin, axis=1, keepdims=True)
    conf_sums = jnp.sum(conf * in_bin, axis=1, keepdims=True)
    partial = jnp.concatenate([counts, acc_sums, conf_sums], axis=1)

    @pl.when(i == 0)
    def _init():
        acc_scratch[...] = partial

    @pl.when(i != 0)
    def _accum():
        acc_scratch[...] = acc_scratch[...] + partial

    @pl.when(i == g - 1)
    def _finalize():
        tot = acc_scratch[...]
        count = tot[:, 0:1]
        acc_sum = tot[:, 1:2]
        conf_sum = tot[:, 2:3]
        prop = count / float(n_rows)
        safe = jnp.maximum(count, 1.0)
        acc_bin = acc_sum / safe
        conf_bin = conf_sum / safe
        nonempty = count > 0.0
        gaps = jnp.where(nonempty, jnp.abs(conf_bin - acc_bin) * prop, 0.0)
        ece_ref[...] = jnp.sum(gaps, keepdims=True)
        accs_ref[...] = jnp.where(nonempty, acc_bin, jnp.nan)
        confs_ref[...] = jnp.where(nonempty, conf_bin, jnp.nan)


@jax.jit
def kernel(logits, labels):
    n_rows, n_cols = logits.shape
    lt = logits.T
    block_c = _BLOCK_C
    grid = n_rows // block_c
    labels3 = labels.reshape(grid, 1, block_c)

    ece, accs, confs = pl.pallas_call(
        functools.partial(_tc_kernel, n_rows=n_rows, block_c=block_c),
        grid=(grid,),
        in_specs=[
            pl.BlockSpec((n_cols, block_c), lambda i: (0, i)),
            pl.BlockSpec((1, 1, block_c), lambda i: (i, 0, 0)),
        ],
        out_specs=[
            pl.BlockSpec((1, 1), lambda i: (0, 0)),
            pl.BlockSpec((_N_BINS, 1), lambda i: (0, 0)),
            pl.BlockSpec((_N_BINS, 1), lambda i: (0, 0)),
        ],
        out_shape=[
            jax.ShapeDtypeStruct((1, 1), jnp.float32),
            jax.ShapeDtypeStruct((_N_BINS, 1), jnp.float32),
            jax.ShapeDtypeStruct((_N_BINS, 1), jnp.float32),
        ],
        scratch_shapes=[pltpu.VMEM((_N_BINS, 3), jnp.float32)],
    )(lt, labels3)
    return ece.reshape(1), accs.reshape(_N_BINS), confs.reshape(_N_BINS)
